# baseline (speedup 1.0000x reference)
import functools

import jax
import jax.numpy as jnp
from jax import lax
from jax.experimental import pallas as pl
from jax.experimental.pallas import tpu as pltpu
from jax.experimental.pallas import tpu_sc as plsc

_BT = 512
_CT = 2048


def _argmin_body(x_ref, cbt_ref, x2_ref, c2_ref, idx_ref):
    xb = x_ref[...].astype(jnp.bfloat16)
    x2 = x2_ref[...]
    K = cbt_ref.shape[1]
    BT = xb.shape[0]

    def step(t, carry):
        rmin, rarg = carry
        ct = cbt_ref[:, pl.ds(t * _CT, _CT)].astype(jnp.bfloat16)
        c2 = c2_ref[:, pl.ds(t * _CT, _CT)]
        mm = lax.dot_general(xb, ct, (((1,), (0,)), ((), ())),
                             preferred_element_type=jnp.float32)
        s = (x2 + c2) - 2.0 * mm
        tmin = jnp.min(s, axis=1, keepdims=True)
        ids = lax.broadcasted_iota(jnp.int32, s.shape, 1) + t * _CT
        targ = jnp.min(jnp.where(s == tmin, ids, K), axis=1, keepdims=True)
        rv = rmin.astype(jnp.bfloat16).astype(jnp.float32)
        better = tmin < rv
        return (jnp.where(better, tmin, rv),
                jnp.where(better, targ, rarg))

    rmin0 = jnp.full((BT, 1), jnp.inf, dtype=jnp.float32)
    rarg0 = jnp.zeros((BT, 1), dtype=jnp.int32)
    _, rarg = lax.fori_loop(0, K // _CT, step, (rmin0, rarg0))
    idx_ref[0] = rarg


def _tc_argmin(xf, cbt, x2, c2):
    N, D = xf.shape
    K = cbt.shape[1]
    nb = N // _BT
    out = pl.pallas_call(
        _argmin_body,
        grid=(nb,),
        in_specs=[
            pl.BlockSpec((_BT, D), lambda i: (i, 0)),
            pl.BlockSpec((D, K), lambda i: (0, 0)),
            pl.BlockSpec((_BT, 1), lambda i: (i, 0)),
            pl.BlockSpec((1, K), lambda i: (0, 0)),
        ],
        out_specs=pl.BlockSpec((1, _BT, 1), lambda i: (i, 0, 0)),
        out_shape=jax.ShapeDtypeStruct((nb, _BT, 1), jnp.int32),
    )(xf, cbt, x2, c2)
    return out.reshape(N)


_GC = 128


def _sc_gather_loss(cb, idx, xflat):
    K, D = cb.shape
    N = idx.shape[0]
    info = plsc.get_sparse_core_info()
    NC, NS = info.num_cores, info.num_subcores
    NW = NC * NS
    bpw = N // NW
    nch = bpw // _GC
    idx3 = idx.reshape(NW, nch, _GC)
    mesh = plsc.VectorSubcoreMesh(core_axis_name="c", subcore_axis_name="s")

    @functools.partial(
        pl.kernel,
        mesh=mesh,
        compiler_params=pltpu.CompilerParams(use_tc_tiling_on_sc=False),
        out_type=[
            jax.ShapeDtypeStruct((N, D), jnp.float32),
            jax.ShapeDtypeStruct((NW, 16), jnp.float32),
        ],
        scratch_types=[
            pltpu.VMEM((nch, _GC), jnp.int32),
            pltpu.VMEM((bpw, D), jnp.float32),
            pltpu.VMEM((bpw, D), jnp.float32),
            pltpu.VMEM((16,), jnp.float32),
            pltpu.SemaphoreType.DMA,
        ],
    )
    def sck(cb_hbm, idx_hbm, x_hbm, q_hbm, part_hbm,
            idx_v, rows_v, x_v, acc_v, sem):
        wid = lax.axis_index("s") * NC + lax.axis_index("c")
        base = wid * bpw
        pltpu.sync_copy(idx_hbm.at[wid], idx_v)
        pltpu.sync_copy(x_hbm.at[pl.ds(base, bpw)], x_v)
        handles = [
            pltpu.async_copy(cb_hbm.at[idx_v.at[k]],
                             rows_v.at[pl.ds(k * _GC, _GC)], sem)
            for k in range(nch)
        ]
        for h in handles:
            h.wait()
        pltpu.sync_copy(rows_v, q_hbm.at[pl.ds(base, bpw)])

        def lbody(i, acc):
            for j in range(D // 16):
                dv = rows_v[i, 16 * j:16 * (j + 1)] - x_v[i, 16 * j:16 * (j + 1)]
                acc = acc + dv * dv
            return acc

        acc = lax.fori_loop(0, bpw, lbody, jnp.zeros((16,), jnp.float32))
        acc_v[...] = acc
        pltpu.sync_copy(acc_v, part_hbm.at[wid])

    return sck(cb, idx3, xflat)


def kernel(x, hidden_dim, codebook):
    B, S, D = x.shape
    N = B * S
    flat = x.reshape(N, D)
    x2 = jnp.sum(flat ** 2, axis=1, keepdims=True)
    c2 = jnp.sum(codebook ** 2, axis=1)
    cbt, x2b, c2b = jax.lax.optimization_barrier(
        (codebook.T, x2, c2[None, :]))
    idx = _tc_argmin(flat, cbt, x2b, c2b)
    q_flat, parts = _sc_gather_loss(codebook, idx, flat)
    quantized = q_flat.reshape(x.shape)
    total = jnp.sum(parts) * (1.25 / (N * D))
    dim_ok = hidden_dim == codebook.shape[1]
    quantized = jnp.where(dim_ok, quantized, jnp.nan)
    total = jnp.where(dim_ok, total, jnp.nan)
    return quantized, total


# speedup vs baseline: 1.0258x; 1.0258x over previous
import functools

import jax
import jax.numpy as jnp
from jax import lax
from jax.experimental import pallas as pl
from jax.experimental.pallas import tpu as pltpu
from jax.experimental.pallas import tpu_sc as plsc

_BT = 1024
_CT = 2048


def _argmin_body(x_ref, cbt_ref, x2_ref, c2_ref, idx_ref):
    xb = x_ref[...].astype(jnp.bfloat16)
    x2 = x2_ref[...]
    K = cbt_ref.shape[1]
    BT = xb.shape[0]

    def step(t, carry):
        rmin, rarg = carry
        ct = cbt_ref[:, pl.ds(t * _CT, _CT)].astype(jnp.bfloat16)
        c2 = c2_ref[:, pl.ds(t * _CT, _CT)]
        mm = lax.dot_general(xb, ct, (((1,), (0,)), ((), ())),
                             preferred_element_type=jnp.float32)
        s = (x2 + c2) - 2.0 * mm
        tmin = jnp.min(s, axis=1, keepdims=True)
        ids = lax.broadcasted_iota(jnp.int32, s.shape, 1) + t * _CT
        targ = jnp.min(jnp.where(s == tmin, ids, K), axis=1, keepdims=True)
        rv = rmin.astype(jnp.bfloat16).astype(jnp.float32)
        better = tmin < rv
        return (jnp.where(better, tmin, rv),
                jnp.where(better, targ, rarg))

    rmin0 = jnp.full((BT, 1), jnp.inf, dtype=jnp.float32)
    rarg0 = jnp.zeros((BT, 1), dtype=jnp.int32)
    _, rarg = lax.fori_loop(0, K // _CT, step, (rmin0, rarg0))
    idx_ref[0] = rarg


def _tc_argmin(xf, cbt, x2, c2):
    N, D = xf.shape
    K = cbt.shape[1]
    nb = N // _BT
    out = pl.pallas_call(
        _argmin_body,
        grid=(nb,),
        in_specs=[
            pl.BlockSpec((_BT, D), lambda i: (i, 0)),
            pl.BlockSpec((D, K), lambda i: (0, 0)),
            pl.BlockSpec((_BT, 1), lambda i: (i, 0)),
            pl.BlockSpec((1, K), lambda i: (0, 0)),
        ],
        out_specs=pl.BlockSpec((1, _BT, 1), lambda i: (i, 0, 0)),
        out_shape=jax.ShapeDtypeStruct((nb, _BT, 1), jnp.int32),
    )(xf, cbt, x2, c2)
    return out.reshape(N)


_GC = 128


def _sc_gather_loss(cb, idx, xflat):
    K, D = cb.shape
    N = idx.shape[0]
    info = plsc.get_sparse_core_info()
    NC, NS = info.num_cores, info.num_subcores
    NW = NC * NS
    bpw = N // NW
    nch = bpw // _GC
    idx3 = idx.reshape(NW, nch, _GC)
    mesh = plsc.VectorSubcoreMesh(core_axis_name="c", subcore_axis_name="s")

    @functools.partial(
        pl.kernel,
        mesh=mesh,
        compiler_params=pltpu.CompilerParams(use_tc_tiling_on_sc=False),
        out_type=[
            jax.ShapeDtypeStruct((N, D), jnp.float32),
            jax.ShapeDtypeStruct((NW, 16), jnp.float32),
        ],
        scratch_types=[
            pltpu.VMEM((nch, _GC), jnp.int32),
            pltpu.VMEM((bpw, D), jnp.float32),
            pltpu.VMEM((bpw, D), jnp.float32),
            pltpu.VMEM((16,), jnp.float32),
            pltpu.SemaphoreType.DMA,
        ],
    )
    def sck(cb_hbm, idx_hbm, x_hbm, q_hbm, part_hbm,
            idx_v, rows_v, x_v, acc_v, sem):
        wid = lax.axis_index("s") * NC + lax.axis_index("c")
        base = wid * bpw
        pltpu.sync_copy(idx_hbm.at[wid], idx_v)
        pltpu.sync_copy(x_hbm.at[pl.ds(base, bpw)], x_v)
        handles = [
            pltpu.async_copy(cb_hbm.at[idx_v.at[k]],
                             rows_v.at[pl.ds(k * _GC, _GC)], sem)
            for k in range(nch)
        ]
        for h in handles:
            h.wait()
        pltpu.sync_copy(rows_v, q_hbm.at[pl.ds(base, bpw)])

        def lbody(i, acc):
            for j in range(D // 16):
                dv = rows_v[i, 16 * j:16 * (j + 1)] - x_v[i, 16 * j:16 * (j + 1)]
                acc = acc + dv * dv
            return acc

        acc = lax.fori_loop(0, bpw, lbody, jnp.zeros((16,), jnp.float32))
        acc_v[...] = acc
        pltpu.sync_copy(acc_v, part_hbm.at[wid])

    return sck(cb, idx3, xflat)


def kernel(x, hidden_dim, codebook):
    B, S, D = x.shape
    N = B * S
    flat = x.reshape(N, D)
    x2 = jnp.sum(flat ** 2, axis=1, keepdims=True)
    c2 = jnp.sum(codebook ** 2, axis=1)
    cbt, x2b, c2b = jax.lax.optimization_barrier(
        (codebook.T, x2, c2[None, :]))
    idx = _tc_argmin(flat, cbt, x2b, c2b)
    q_flat, parts = _sc_gather_loss(codebook, idx, flat)
    quantized = q_flat.reshape(x.shape)
    total = jnp.sum(parts) * (1.25 / (N * D))
    dim_ok = hidden_dim == codebook.shape[1]
    quantized = jnp.where(dim_ok, quantized, jnp.nan)
    total = jnp.where(dim_ok, total, jnp.nan)
    return quantized, total


# speedup vs baseline: 1.0389x; 1.0127x over previous
import functools

import jax
import jax.numpy as jnp
from jax import lax
from jax.experimental import pallas as pl
from jax.experimental.pallas import tpu as pltpu
from jax.experimental.pallas import tpu_sc as plsc

_BT = 2048
_CT = 2048


def _argmin_body(x_ref, cbt_ref, x2_ref, c2_ref, idx_ref):
    xb = x_ref[...].astype(jnp.bfloat16)
    x2 = x2_ref[...]
    K = cbt_ref.shape[1]
    BT = xb.shape[0]

    def step(t, carry):
        rmin, rarg = carry
        ct = cbt_ref[:, pl.ds(t * _CT, _CT)].astype(jnp.bfloat16)
        c2 = c2_ref[:, pl.ds(t * _CT, _CT)]
        mm = lax.dot_general(xb, ct, (((1,), (0,)), ((), ())),
                             preferred_element_type=jnp.float32)
        s = (x2 + c2) - 2.0 * mm
        tmin = jnp.min(s, axis=1, keepdims=True)
        ids = lax.broadcasted_iota(jnp.int32, s.shape, 1) + t * _CT
        targ = jnp.min(jnp.where(s == tmin, ids, K), axis=1, keepdims=True)
        rv = rmin.astype(jnp.bfloat16).astype(jnp.float32)
        better = tmin < rv
        return (jnp.where(better, tmin, rv),
                jnp.where(better, targ, rarg))

    rmin0 = jnp.full((BT, 1), jnp.inf, dtype=jnp.float32)
    rarg0 = jnp.zeros((BT, 1), dtype=jnp.int32)
    _, rarg = lax.fori_loop(0, K // _CT, step, (rmin0, rarg0))
    idx_ref[0] = rarg


def _tc_argmin(xf, cbt, x2, c2):
    N, D = xf.shape
    K = cbt.shape[1]
    nb = N // _BT
    out = pl.pallas_call(
        _argmin_body,
        grid=(nb,),
        in_specs=[
            pl.BlockSpec((_BT, D), lambda i: (i, 0)),
            pl.BlockSpec((D, K), lambda i: (0, 0)),
            pl.BlockSpec((_BT, 1), lambda i: (i, 0)),
            pl.BlockSpec((1, K), lambda i: (0, 0)),
        ],
        out_specs=pl.BlockSpec((1, _BT, 1), lambda i: (i, 0, 0)),
        out_shape=jax.ShapeDtypeStruct((nb, _BT, 1), jnp.int32),
    )(xf, cbt, x2, c2)
    return out.reshape(N)


_GC = 128


def _sc_gather_loss(cb, idx, xflat):
    K, D = cb.shape
    N = idx.shape[0]
    info = plsc.get_sparse_core_info()
    NC, NS = info.num_cores, info.num_subcores
    NW = NC * NS
    bpw = N // NW
    nch = bpw // _GC
    idx3 = idx.reshape(NW, nch, _GC)
    mesh = plsc.VectorSubcoreMesh(core_axis_name="c", subcore_axis_name="s")

    @functools.partial(
        pl.kernel,
        mesh=mesh,
        compiler_params=pltpu.CompilerParams(use_tc_tiling_on_sc=False),
        out_type=[
            jax.ShapeDtypeStruct((N, D), jnp.float32),
            jax.ShapeDtypeStruct((NW, 16), jnp.float32),
        ],
        scratch_types=[
            pltpu.VMEM((nch, _GC), jnp.int32),
            pltpu.VMEM((bpw, D), jnp.float32),
            pltpu.VMEM((bpw, D), jnp.float32),
            pltpu.VMEM((16,), jnp.float32),
            pltpu.SemaphoreType.DMA,
        ],
    )
    def sck(cb_hbm, idx_hbm, x_hbm, q_hbm, part_hbm,
            idx_v, rows_v, x_v, acc_v, sem):
        wid = lax.axis_index("s") * NC + lax.axis_index("c")
        base = wid * bpw
        pltpu.sync_copy(idx_hbm.at[wid], idx_v)
        pltpu.sync_copy(x_hbm.at[pl.ds(base, bpw)], x_v)
        handles = [
            pltpu.async_copy(cb_hbm.at[idx_v.at[k]],
                             rows_v.at[pl.ds(k * _GC, _GC)], sem)
            for k in range(nch)
        ]
        for h in handles:
            h.wait()
        pltpu.sync_copy(rows_v, q_hbm.at[pl.ds(base, bpw)])

        def lbody(i, acc):
            for j in range(D // 16):
                dv = rows_v[i, 16 * j:16 * (j + 1)] - x_v[i, 16 * j:16 * (j + 1)]
                acc = acc + dv * dv
            return acc

        acc = lax.fori_loop(0, bpw, lbody, jnp.zeros((16,), jnp.float32))
        acc_v[...] = acc
        pltpu.sync_copy(acc_v, part_hbm.at[wid])

    return sck(cb, idx3, xflat)


def kernel(x, hidden_dim, codebook):
    B, S, D = x.shape
    N = B * S
    flat = x.reshape(N, D)
    x2 = jnp.sum(flat ** 2, axis=1, keepdims=True)
    c2 = jnp.sum(codebook ** 2, axis=1)
    cbt, x2b, c2b = jax.lax.optimization_barrier(
        (codebook.T, x2, c2[None, :]))
    idx = _tc_argmin(flat, cbt, x2b, c2b)
    q_flat, parts = _sc_gather_loss(codebook, idx, flat)
    quantized = q_flat.reshape(x.shape)
    total = jnp.sum(parts) * (1.25 / (N * D))
    dim_ok = hidden_dim == codebook.shape[1]
    quantized = jnp.where(dim_ok, quantized, jnp.nan)
    total = jnp.where(dim_ok, total, jnp.nan)
    return quantized, total


# speedup vs baseline: 1.0395x; 1.0006x over previous
import functools

import jax
import jax.numpy as jnp
from jax import lax
from jax.experimental import pallas as pl
from jax.experimental.pallas import tpu as pltpu
from jax.experimental.pallas import tpu_sc as plsc

_BT = 2048
_CT = 2048


def _argmin_body(x_ref, cbt_ref, x2_ref, c2_ref, idx_ref):
    xb = x_ref[...].astype(jnp.bfloat16)
    x2 = x2_ref[...]
    K = cbt_ref.shape[1]
    BT = xb.shape[0]

    def step(t, carry):
        rmin, rarg = carry
        ct = cbt_ref[:, pl.ds(t * _CT, _CT)].astype(jnp.bfloat16)
        c2 = c2_ref[:, pl.ds(t * _CT, _CT)]
        mm = lax.dot_general(xb, ct, (((1,), (0,)), ((), ())),
                             preferred_element_type=jnp.float32)
        s = (x2 + c2) - 2.0 * mm
        tmin = jnp.min(s, axis=1, keepdims=True)
        ids = lax.broadcasted_iota(jnp.int32, s.shape, 1)
        targ = jnp.min(jnp.where(s == tmin, ids, K), axis=1,
                       keepdims=True) + t * _CT
        rv = rmin.astype(jnp.bfloat16).astype(jnp.float32)
        better = tmin < rv
        return (jnp.where(better, tmin, rv),
                jnp.where(better, targ, rarg))

    rmin0 = jnp.full((BT, 1), jnp.inf, dtype=jnp.float32)
    rarg0 = jnp.zeros((BT, 1), dtype=jnp.int32)
    _, rarg = lax.fori_loop(0, K // _CT, step, (rmin0, rarg0))
    idx_ref[0] = rarg


def _tc_argmin(xf, cbt, x2, c2):
    N, D = xf.shape
    K = cbt.shape[1]
    nb = N // _BT
    out = pl.pallas_call(
        _argmin_body,
        grid=(nb,),
        in_specs=[
            pl.BlockSpec((_BT, D), lambda i: (i, 0)),
            pl.BlockSpec((D, K), lambda i: (0, 0)),
            pl.BlockSpec((_BT, 1), lambda i: (i, 0)),
            pl.BlockSpec((1, K), lambda i: (0, 0)),
        ],
        out_specs=pl.BlockSpec((1, _BT, 1), lambda i: (i, 0, 0)),
        out_shape=jax.ShapeDtypeStruct((nb, _BT, 1), jnp.int32),
    )(xf, cbt, x2, c2)
    return out.reshape(N)


_GC = 128


def _sc_gather_loss(cb, idx, xflat):
    K, D = cb.shape
    N = idx.shape[0]
    info = plsc.get_sparse_core_info()
    NC, NS = info.num_cores, info.num_subcores
    NW = NC * NS
    bpw = N // NW
    nch = bpw // _GC
    idx3 = idx.reshape(NW, nch, _GC)
    mesh = plsc.VectorSubcoreMesh(core_axis_name="c", subcore_axis_name="s")

    @functools.partial(
        pl.kernel,
        mesh=mesh,
        compiler_params=pltpu.CompilerParams(use_tc_tiling_on_sc=False),
        out_type=[
            jax.ShapeDtypeStruct((N, D), jnp.float32),
            jax.ShapeDtypeStruct((NW, 16), jnp.float32),
        ],
        scratch_types=[
            pltpu.VMEM((nch, _GC), jnp.int32),
            pltpu.VMEM((bpw, D), jnp.float32),
            pltpu.VMEM((bpw, D), jnp.float32),
            pltpu.VMEM((16,), jnp.float32),
            pltpu.SemaphoreType.DMA,
        ],
    )
    def sck(cb_hbm, idx_hbm, x_hbm, q_hbm, part_hbm,
            idx_v, rows_v, x_v, acc_v, sem):
        wid = lax.axis_index("s") * NC + lax.axis_index("c")
        base = wid * bpw
        pltpu.sync_copy(idx_hbm.at[wid], idx_v)
        pltpu.sync_copy(x_hbm.at[pl.ds(base, bpw)], x_v)
        handles = [
            pltpu.async_copy(cb_hbm.at[idx_v.at[k]],
                             rows_v.at[pl.ds(k * _GC, _GC)], sem)
            for k in range(nch)
        ]
        for h in handles:
            h.wait()
        pltpu.sync_copy(rows_v, q_hbm.at[pl.ds(base, bpw)])

        def lbody(i, acc):
            for j in range(D // 16):
                dv = rows_v[i, 16 * j:16 * (j + 1)] - x_v[i, 16 * j:16 * (j + 1)]
                acc = acc + dv * dv
            return acc

        acc = lax.fori_loop(0, bpw, lbody, jnp.zeros((16,), jnp.float32))
        acc_v[...] = acc
        pltpu.sync_copy(acc_v, part_hbm.at[wid])

    return sck(cb, idx3, xflat)


def kernel(x, hidden_dim, codebook):
    B, S, D = x.shape
    N = B * S
    flat = x.reshape(N, D)
    x2 = jnp.sum(flat ** 2, axis=1, keepdims=True)
    c2 = jnp.sum(codebook ** 2, axis=1)
    cbt, x2b, c2b = jax.lax.optimization_barrier(
        (codebook.T, x2, c2[None, :]))
    idx = _tc_argmin(flat, cbt, x2b, c2b)
    q_flat, parts = _sc_gather_loss(codebook, idx, flat)
    quantized = q_flat.reshape(x.shape)
    total = jnp.sum(parts) * (1.25 / (N * D))
    dim_ok = hidden_dim == codebook.shape[1]
    quantized = jnp.where(dim_ok, quantized, jnp.nan)
    total = jnp.where(dim_ok, total, jnp.nan)
    return quantized, total
